# baseline (speedup 1.0000x reference)
import jax
import jax.numpy as jnp
from jax import lax
from jax.experimental import pallas as pl
from jax.experimental.pallas import tpu as pltpu
from jax.experimental.pallas import tpu_sc as plsc

NC, NS = 2, 16
NW = NC * NS
B = 16384
D = 128
NL, NSUBJ = 30, 100
CHUNK = 64
CPW = B // (NW * CHUNK)
NBUF = 4


def _body(labi, subi, labt, subt, out, idxL, idxS, tabL, tabS, bufL, bufS,
          gsemL, gsemS, wsemL, wsemS):
    sid = lax.axis_index("s")
    wid = sid * NC + lax.axis_index("c")
    row0 = wid * CPW
    base = row0 * CHUNK

    @pl.when(sid == 0)
    def _stage():
        cL = pltpu.async_copy(labt, tabL, gsemL)
        cS = pltpu.async_copy(subt, tabS, gsemS)
        cL.wait()
        cS.wait()

    ic = []
    for j in range(CPW):
        ic.append(pltpu.async_copy(labi.at[pl.ds(base + j * CHUNK, CHUNK)],
                                   idxL.at[j], wsemL))
        ic.append(pltpu.async_copy(subi.at[pl.ds(base + j * CHUNK, CHUNK)],
                                   idxS.at[j], wsemS))
    for c in ic:
        c.wait()
    plsc.subcore_barrier()

    gl = [None] * CPW
    gs = [None] * CPW
    wl = [None] * CPW
    ws = [None] * CPW
    gl[0] = pltpu.async_copy(tabL.at[idxL.at[0]], bufL.at[0], gsemL)
    gs[0] = pltpu.async_copy(tabS.at[idxS.at[0]], bufS.at[0], gsemS)
    for j in range(CPW):
        gl[j].wait()
        gs[j].wait()
        r = (row0 + j) * CHUNK
        wl[j] = pltpu.async_copy(bufL.at[j % NBUF],
                                 out.at[pl.ds(r, CHUNK), pl.ds(0, D)], wsemL)
        ws[j] = pltpu.async_copy(bufS.at[j % NBUF],
                                 out.at[pl.ds(r, CHUNK), pl.ds(D, D)], wsemS)
        if j + 1 < CPW:
            if j + 1 >= NBUF:
                wl[j + 1 - NBUF].wait()
                ws[j + 1 - NBUF].wait()
            gl[j + 1] = pltpu.async_copy(tabL.at[idxL.at[j + 1]],
                                         bufL.at[(j + 1) % NBUF], gsemL)
            gs[j + 1] = pltpu.async_copy(tabS.at[idxS.at[j + 1]],
                                         bufS.at[(j + 1) % NBUF], gsemS)
    for j in range(max(0, CPW - NBUF), CPW):
        wl[j].wait()
        ws[j].wait()


def kernel(lab_ids, subject_ids, lab_table, subject_table):
    labi = lab_ids.astype(jnp.int32)
    subi = subject_ids.astype(jnp.int32)
    mesh = plsc.VectorSubcoreMesh(core_axis_name="c", subcore_axis_name="s")
    f = pl.kernel(
        _body,
        mesh=mesh,
        out_type=jax.ShapeDtypeStruct((B, 2 * D), jnp.float32),
        scratch_types=[
            pltpu.VMEM((CPW, CHUNK), jnp.int32),
            pltpu.VMEM((CPW, CHUNK), jnp.int32),
            pltpu.VMEM_SHARED((NL, D), jnp.float32),
            pltpu.VMEM_SHARED((NSUBJ, D), jnp.float32),
            pltpu.VMEM((NBUF, CHUNK, D), jnp.float32),
            pltpu.VMEM((NBUF, CHUNK, D), jnp.float32),
            pltpu.SemaphoreType.DMA,
            pltpu.SemaphoreType.DMA,
            pltpu.SemaphoreType.DMA,
            pltpu.SemaphoreType.DMA,
        ],
    )
    return f(labi, subi, lab_table, subject_table)


# speedup vs baseline: 1.0218x; 1.0218x over previous
import jax
import jax.numpy as jnp
from jax import lax
from jax.experimental import pallas as pl
from jax.experimental.pallas import tpu as pltpu
from jax.experimental.pallas import tpu_sc as plsc

NC, NS = 2, 16
NW = NC * NS
B = 16384
D = 128
NL, NSUBJ = 30, 100
CHUNK = 128
CPW = B // (NW * CHUNK)
NBUF = 3


def _body(labi, subi, labt, subt, out, idxL, idxS, tabL, tabS, bufL, bufS,
          gsemL, gsemS, wsemL, wsemS):
    sid = lax.axis_index("s")
    wid = sid * NC + lax.axis_index("c")
    row0 = wid * CPW
    base = row0 * CHUNK

    @pl.when(sid == 0)
    def _stage():
        cL = pltpu.async_copy(labt, tabL, gsemL)
        cS = pltpu.async_copy(subt, tabS, gsemS)
        cL.wait()
        cS.wait()

    ic = []
    for j in range(CPW):
        ic.append(pltpu.async_copy(labi.at[pl.ds(base + j * CHUNK, CHUNK)],
                                   idxL.at[j], wsemL))
        ic.append(pltpu.async_copy(subi.at[pl.ds(base + j * CHUNK, CHUNK)],
                                   idxS.at[j], wsemS))
    for c in ic:
        c.wait()
    plsc.subcore_barrier()

    gl = [None] * CPW
    gs = [None] * CPW
    wl = [None] * CPW
    ws = [None] * CPW
    gl[0] = pltpu.async_copy(tabL.at[idxL.at[0]], bufL.at[0], gsemL)
    gs[0] = pltpu.async_copy(tabS.at[idxS.at[0]], bufS.at[0], gsemS)
    for j in range(CPW):
        gl[j].wait()
        gs[j].wait()
        r = (row0 + j) * CHUNK
        wl[j] = pltpu.async_copy(bufL.at[j % NBUF],
                                 out.at[pl.ds(r, CHUNK), pl.ds(0, D)], wsemL)
        ws[j] = pltpu.async_copy(bufS.at[j % NBUF],
                                 out.at[pl.ds(r, CHUNK), pl.ds(D, D)], wsemS)
        if j + 1 < CPW:
            if j + 1 >= NBUF:
                wl[j + 1 - NBUF].wait()
                ws[j + 1 - NBUF].wait()
            gl[j + 1] = pltpu.async_copy(tabL.at[idxL.at[j + 1]],
                                         bufL.at[(j + 1) % NBUF], gsemL)
            gs[j + 1] = pltpu.async_copy(tabS.at[idxS.at[j + 1]],
                                         bufS.at[(j + 1) % NBUF], gsemS)
    for j in range(max(0, CPW - NBUF), CPW):
        wl[j].wait()
        ws[j].wait()


def kernel(lab_ids, subject_ids, lab_table, subject_table):
    labi = lab_ids.astype(jnp.int32)
    subi = subject_ids.astype(jnp.int32)
    mesh = plsc.VectorSubcoreMesh(core_axis_name="c", subcore_axis_name="s")
    f = pl.kernel(
        _body,
        mesh=mesh,
        out_type=jax.ShapeDtypeStruct((B, 2 * D), jnp.float32),
        scratch_types=[
            pltpu.VMEM((CPW, CHUNK), jnp.int32),
            pltpu.VMEM((CPW, CHUNK), jnp.int32),
            pltpu.VMEM_SHARED((NL, D), jnp.float32),
            pltpu.VMEM_SHARED((NSUBJ, D), jnp.float32),
            pltpu.VMEM((NBUF, CHUNK, D), jnp.float32),
            pltpu.VMEM((NBUF, CHUNK, D), jnp.float32),
            pltpu.SemaphoreType.DMA,
            pltpu.SemaphoreType.DMA,
            pltpu.SemaphoreType.DMA,
            pltpu.SemaphoreType.DMA,
        ],
    )
    return f(labi, subi, lab_table, subject_table)
